# baseline (speedup 1.0000x reference)
import functools

import jax
import jax.numpy as jnp
from jax import lax
from jax.experimental import pallas as pl
from jax.experimental.pallas import tpu as pltpu
from jax.experimental.pallas import tpu_sc as plsc

B, F, V, D = 16384, 66, 100000, 4
H1, H2 = 16, 8
RW = 8
XW = F * RW
NI = B * F
NC, NS = 2, 16
NW = NC * NS
PER_W = NI // NW
IPG = 128
G = 24
STEP = G * IPG
NSTEP = PER_W // STEP
IDX_ROWS = NI // IPG

TB_CH = 2000
TB_NCH = V // TB_CH
TB_IT = -(-TB_NCH // NW)

_MESH = dict(core_axis_name="c", subcore_axis_name="s")
_CP = pltpu.CompilerParams(use_tc_tiling_on_sc=False, needs_layout_passes=False)


def _sc_build_tab(emb_wT, emb_b):
    @functools.partial(
        pl.kernel, mesh=plsc.VectorSubcoreMesh(**_MESH),
        compiler_params=_CP,
        out_type=jax.ShapeDtypeStruct((V, RW), jnp.float32),
        scratch_types=[
            pltpu.VMEM((D, TB_CH), jnp.float32),
            pltpu.VMEM((TB_CH,), jnp.float32),
            pltpu.VMEM((TB_CH, RW), jnp.float32),
        ],
    )
    def k(wT_hbm, b_hbm, tab_hbm, w_v, b_v, rows_v):
        wid = lax.axis_index("s") * NC + lax.axis_index("c")
        for it in range(TB_IT):
            c = wid + NW * it

            @pl.when(c < TB_NCH)
            def _():
                r0 = c * TB_CH
                pltpu.sync_copy(wT_hbm.at[:, pl.ds(r0, TB_CH)], w_v)
                pltpu.sync_copy(b_hbm.at[pl.ds(r0, TB_CH)], b_v)

                def body(g, carry):
                    rows = jnp.arange(16, dtype=jnp.int32) + g * 16
                    zeros = jnp.zeros((16,), jnp.float32)
                    for d in range(D):
                        plsc.store_scatter(
                            rows_v,
                            [rows, jnp.full((16,), d, jnp.int32)],
                            w_v.at[d][pl.ds(g * 16, 16)])
                    plsc.store_scatter(
                        rows_v,
                        [rows, jnp.full((16,), D, jnp.int32)],
                        b_v[pl.ds(g * 16, 16)])
                    for z in range(D + 1, RW):
                        plsc.store_scatter(
                            rows_v,
                            [rows, jnp.full((16,), z, jnp.int32)],
                            zeros)
                    return carry

                lax.fori_loop(0, TB_CH // 16, body, 0)
                pltpu.sync_copy(rows_v, tab_hbm.at[pl.ds(r0, TB_CH)])

    return k(emb_wT, emb_b)


NSPLIT = 2
NI_S = NI // NSPLIT
PER_W_S = NI_S // NW
NSTEP_S = PER_W_S // STEP
GT = (PER_W_S - NSTEP_S * STEP) // IPG


def _sc_gather(tab, idx_flat, base):
    @functools.partial(
        pl.kernel, mesh=plsc.VectorSubcoreMesh(**_MESH),
        compiler_params=_CP,
        out_type=jax.ShapeDtypeStruct((NI_S, RW), jnp.float32),
        scratch_types=[
            pltpu.VMEM((STEP,), jnp.int32),
            pltpu.VMEM((STEP, RW), jnp.float32),
            pltpu.SemaphoreType.DMA,
        ],
    )
    def k(tab_hbm, idx_hbm, out_hbm, idx_v, rows_v, sem):
        wid = lax.axis_index("s") * NC + lax.axis_index("c")

        def batch(o0, n):
            pltpu.sync_copy(idx_hbm.at[pl.ds(base + wid * PER_W_S + o0,
                                             n * IPG)],
                            idx_v.at[pl.ds(0, n * IPG)])
            copies = [
                pltpu.async_copy(tab_hbm.at[idx_v.at[pl.ds(j * IPG, IPG)]],
                                 rows_v.at[pl.ds(j * IPG, IPG)], sem)
                for j in range(n)
            ]
            for cp in copies:
                cp.wait()
            pltpu.sync_copy(rows_v.at[pl.ds(0, n * IPG)],
                            out_hbm.at[pl.ds(wid * PER_W_S + o0, n * IPG)])

        def body(c, carry):
            batch(c * STEP, G)
            return carry

        lax.fori_loop(0, NSTEP_S, body, 0)
        batch(NSTEP_S * STEP, GT)

    return k(tab, idx_flat)


BLK = 2048
NB = B // BLK


def _mlp_body(x_ref, w1_ref, b1_ref, w2_ref, b2_ref, w3_ref, out_ref):
    y = jnp.dot(x_ref[...], w1_ref[...], preferred_element_type=jnp.float32)
    col = lax.broadcasted_iota(jnp.int32, (BLK, 128), 1)
    h1 = jnp.where(col < H1, jnp.maximum(y + b1_ref[...], 0.0), 0.0)
    y2 = jnp.dot(h1, w2_ref[...], preferred_element_type=jnp.float32)
    h2 = jnp.where(col < H2, jnp.maximum(y2 + b2_ref[...], 0.0),
                   jnp.where(col == H2, 1.0, 0.0))
    y3 = jnp.dot(h2, w3_ref[...], preferred_element_type=jnp.float32)
    out_ref[...] = y3[:, 0] + y[:, H1]


def _tc_mlp(x, w1p, b1p, w2p, b2p, w3p):
    rows = x.shape[0]
    return pl.pallas_call(
        _mlp_body,
        grid=(rows // BLK,),
        in_specs=[
            pl.BlockSpec((BLK, XW), lambda i: (i, 0)),
            pl.BlockSpec((XW, 128), lambda i: (0, 0)),
            pl.BlockSpec((1, 128), lambda i: (0, 0)),
            pl.BlockSpec((128, 128), lambda i: (0, 0)),
            pl.BlockSpec((1, 128), lambda i: (0, 0)),
            pl.BlockSpec((128, 128), lambda i: (0, 0)),
        ],
        out_specs=pl.BlockSpec((BLK,), lambda i: (i,)),
        out_shape=jax.ShapeDtypeStruct((rows,), jnp.float32),
    )(x, w1p, b1p, w2p, b2p, w3p)


def _pack_weights(W1, b1, W2, b2, W3, b3):
    w1r = jnp.transpose(W1.reshape(H1, F, D), (1, 2, 0))
    w1p = (jnp.zeros((F, RW, 128), jnp.float32)
           .at[:, :D, :H1].set(w1r)
           .at[:, D, H1].set(1.0)
           .reshape(XW, 128))
    b1p = jnp.zeros((1, 128), jnp.float32).at[0, :H1].set(b1)
    w2p = jnp.zeros((128, 128), jnp.float32).at[:H1, :H2].set(W2.T)
    b2p = jnp.zeros((1, 128), jnp.float32).at[0, :H2].set(b2)
    w3p = (jnp.zeros((128, 128), jnp.float32)
           .at[:H2, 0].set(W3[0]).at[H2, 0].set(b3[0]))
    return w1p, b1p, w2p, b2p, w3p


def kernel(fids_batch, emb_w, emb_b, W1, b1, W2, b2, W3, b3):
    tab = _sc_build_tab(emb_w.T, emb_b)
    idx_flat = fids_batch.reshape(NI)
    w1p, b1p, w2p, b2p, w3p = _pack_weights(W1, b1, W2, b2, W3, b3)
    gs = [_sc_gather(tab, idx_flat, h * NI_S) for h in range(NSPLIT)]
    preds = []
    for h in range(NSPLIT):
        g = gs[h]
        if preds:
            prev, g = lax.optimization_barrier((preds[-1], g))
            preds[-1] = prev
        x = g.reshape(B // NSPLIT, XW)
        preds.append(_tc_mlp(x, w1p, b1p, w2p, b2p, w3p))
    return jnp.concatenate(preds)


# speedup vs baseline: 2.8136x; 2.8136x over previous
import functools

import jax
import jax.numpy as jnp
from jax import lax
from jax.experimental import pallas as pl
from jax.experimental.pallas import tpu as pltpu
from jax.experimental.pallas import tpu_sc as plsc

B, F, V, D = 16384, 66, 100000, 4
H1, H2 = 16, 8
RW = 8
XW = F * RW
NI = B * F
NC, NS = 2, 16
NW = NC * NS
PER_W = NI // NW
IPG = 128
G = 24
STEP = G * IPG
NSTEP = PER_W // STEP
IDX_ROWS = NI // IPG

TB_CH = 2000
TB_NCH = V // TB_CH
TB_IT = -(-TB_NCH // NW)

_MESH = dict(core_axis_name="c", subcore_axis_name="s")
_CP = pltpu.CompilerParams(use_tc_tiling_on_sc=False, needs_layout_passes=False)


def _sc_build_tab(emb_wT, emb_b):
    @functools.partial(
        pl.kernel, mesh=plsc.VectorSubcoreMesh(**_MESH),
        compiler_params=_CP,
        out_type=jax.ShapeDtypeStruct((V, RW), jnp.float32),
        scratch_types=[
            pltpu.VMEM((D, TB_CH), jnp.float32),
            pltpu.VMEM((TB_CH,), jnp.float32),
            pltpu.VMEM((TB_CH, RW), jnp.float32),
        ],
    )
    def k(wT_hbm, b_hbm, tab_hbm, w_v, b_v, rows_v):
        wid = lax.axis_index("s") * NC + lax.axis_index("c")
        for it in range(TB_IT):
            c = wid + NW * it

            @pl.when(c < TB_NCH)
            def _():
                r0 = c * TB_CH
                pltpu.sync_copy(wT_hbm.at[:, pl.ds(r0, TB_CH)], w_v)
                pltpu.sync_copy(b_hbm.at[pl.ds(r0, TB_CH)], b_v)

                def body(g, carry):
                    rows = jnp.arange(16, dtype=jnp.int32) + g * 16
                    zeros = jnp.zeros((16,), jnp.float32)
                    for d in range(D):
                        plsc.store_scatter(
                            rows_v,
                            [rows, jnp.full((16,), d, jnp.int32)],
                            w_v.at[d][pl.ds(g * 16, 16)])
                    plsc.store_scatter(
                        rows_v,
                        [rows, jnp.full((16,), D, jnp.int32)],
                        b_v[pl.ds(g * 16, 16)])
                    for z in range(D + 1, RW):
                        plsc.store_scatter(
                            rows_v,
                            [rows, jnp.full((16,), z, jnp.int32)],
                            zeros)
                    return carry

                lax.fori_loop(0, TB_CH // 16, body, 0)
                pltpu.sync_copy(rows_v, tab_hbm.at[pl.ds(r0, TB_CH)])

    return k(emb_wT, emb_b)


NSPLIT = 2
NI_S = NI // NSPLIT
PER_W_S = NI_S // NW
NSTEP_S = PER_W_S // STEP
GT = (PER_W_S - NSTEP_S * STEP) // IPG


def _sc_gather(tab, idx_flat, base):
    @functools.partial(
        pl.kernel, mesh=plsc.VectorSubcoreMesh(**_MESH),
        compiler_params=_CP,
        out_type=jax.ShapeDtypeStruct((NI_S, RW), jnp.float32),
        scratch_types=[
            pltpu.VMEM((STEP,), jnp.int32),
            pltpu.VMEM((STEP, RW), jnp.float32),
            pltpu.SemaphoreType.DMA,
        ],
    )
    def k(tab_hbm, idx_hbm, out_hbm, idx_v, rows_v, sem):
        wid = lax.axis_index("s") * NC + lax.axis_index("c")

        def batch(o0, n):
            pltpu.sync_copy(idx_hbm.at[pl.ds(base + wid * PER_W_S + o0,
                                             n * IPG)],
                            idx_v.at[pl.ds(0, n * IPG)])
            copies = [
                pltpu.async_copy(tab_hbm.at[idx_v.at[pl.ds(j * IPG, IPG)]],
                                 rows_v.at[pl.ds(j * IPG, IPG)], sem)
                for j in range(n)
            ]
            for cp in copies:
                cp.wait()
            pltpu.sync_copy(rows_v.at[pl.ds(0, n * IPG)],
                            out_hbm.at[pl.ds(wid * PER_W_S + o0, n * IPG)])

        def body(c, carry):
            batch(c * STEP, G)
            return carry

        lax.fori_loop(0, NSTEP_S, body, 0)
        batch(NSTEP_S * STEP, GT)

    return k(tab, idx_flat)


BLK = 2048
NB = B // BLK


def _mlp_body(x_ref, w1_ref, b1_ref, w2_ref, b2_ref, w3_ref, out_ref):
    y = jnp.dot(x_ref[...], w1_ref[...], preferred_element_type=jnp.float32)
    col = lax.broadcasted_iota(jnp.int32, (BLK, 128), 1)
    h1 = jnp.where(col < H1, jnp.maximum(y + b1_ref[...], 0.0), 0.0)
    y2 = jnp.dot(h1, w2_ref[...], preferred_element_type=jnp.float32)
    h2 = jnp.where(col < H2, jnp.maximum(y2 + b2_ref[...], 0.0),
                   jnp.where(col == H2, 1.0, 0.0))
    y3 = jnp.dot(h2, w3_ref[...], preferred_element_type=jnp.float32)
    out_ref[...] = y3[:, 0] + y[:, H1]


def _tc_mlp(x, w1p, b1p, w2p, b2p, w3p):
    rows = x.shape[0]
    return pl.pallas_call(
        _mlp_body,
        grid=(rows // BLK,),
        in_specs=[
            pl.BlockSpec((BLK, XW), lambda i: (i, 0)),
            pl.BlockSpec((XW, 128), lambda i: (0, 0)),
            pl.BlockSpec((1, 128), lambda i: (0, 0)),
            pl.BlockSpec((128, 128), lambda i: (0, 0)),
            pl.BlockSpec((1, 128), lambda i: (0, 0)),
            pl.BlockSpec((128, 128), lambda i: (0, 0)),
        ],
        out_specs=pl.BlockSpec((BLK,), lambda i: (i,)),
        out_shape=jax.ShapeDtypeStruct((rows,), jnp.float32),
    )(x, w1p, b1p, w2p, b2p, w3p)


def _pack_weights(W1, b1, W2, b2, W3, b3):
    w1r = jnp.transpose(W1.reshape(H1, F, D), (1, 2, 0))
    w1p = (jnp.zeros((F, RW, 128), jnp.float32)
           .at[:, :D, :H1].set(w1r)
           .at[:, D, H1].set(1.0)
           .reshape(XW, 128))
    b1p = jnp.zeros((1, 128), jnp.float32).at[0, :H1].set(b1)
    w2p = jnp.zeros((128, 128), jnp.float32).at[:H1, :H2].set(W2.T)
    b2p = jnp.zeros((1, 128), jnp.float32).at[0, :H2].set(b2)
    w3p = (jnp.zeros((128, 128), jnp.float32)
           .at[:H2, 0].set(W3[0]).at[H2, 0].set(b3[0]))
    return w1p, b1p, w2p, b2p, w3p


def kernel(fids_batch, emb_w, emb_b, W1, b1, W2, b2, W3, b3):
    tab = _sc_build_tab(emb_w.T, emb_b)
    idx_flat = fids_batch.reshape(NI)
    w1p, b1p, w2p, b2p, w3p = _pack_weights(W1, b1, W2, b2, W3, b3)
    preds = []
    for h in range(NSPLIT):
        g = _sc_gather(tab, idx_flat, h * NI_S)
        x = g.reshape(B // NSPLIT, XW)
        preds.append(_tc_mlp(x, w1p, b1p, w2p, b2p, w3p))
    return jnp.concatenate(preds)


# speedup vs baseline: 3.0258x; 1.0754x over previous
import functools

import jax
import jax.numpy as jnp
from jax import lax
from jax.experimental import pallas as pl
from jax.experimental.pallas import tpu as pltpu
from jax.experimental.pallas import tpu_sc as plsc

B, F, V, D = 16384, 66, 100000, 4
H1, H2 = 16, 8
RW = 8
XW = F * RW
NI = B * F
NC, NS = 2, 16
NW = NC * NS
PER_W = NI // NW
IPG = 128
G = 24
STEP = G * IPG
NSTEP = PER_W // STEP
IDX_ROWS = NI // IPG

TB_CH = 2000
TB_NCH = V // TB_CH
TB_IT = -(-TB_NCH // NW)

_MESH = dict(core_axis_name="c", subcore_axis_name="s")
_CP = pltpu.CompilerParams(use_tc_tiling_on_sc=False, needs_layout_passes=False)


def _sc_build_tab(emb_wT, emb_b):
    @functools.partial(
        pl.kernel, mesh=plsc.VectorSubcoreMesh(**_MESH),
        compiler_params=_CP,
        out_type=jax.ShapeDtypeStruct((V, RW), jnp.float32),
        scratch_types=[
            pltpu.VMEM((D, TB_CH), jnp.float32),
            pltpu.VMEM((TB_CH,), jnp.float32),
            pltpu.VMEM((TB_CH, RW), jnp.float32),
        ],
    )
    def k(wT_hbm, b_hbm, tab_hbm, w_v, b_v, rows_v):
        wid = lax.axis_index("s") * NC + lax.axis_index("c")
        for it in range(TB_IT):
            c = wid + NW * it

            @pl.when(c < TB_NCH)
            def _():
                r0 = c * TB_CH
                pltpu.sync_copy(wT_hbm.at[:, pl.ds(r0, TB_CH)], w_v)
                pltpu.sync_copy(b_hbm.at[pl.ds(r0, TB_CH)], b_v)

                def body(g, carry):
                    rows = jnp.arange(16, dtype=jnp.int32) + g * 16
                    zeros = jnp.zeros((16,), jnp.float32)
                    for d in range(D):
                        plsc.store_scatter(
                            rows_v,
                            [rows, jnp.full((16,), d, jnp.int32)],
                            w_v.at[d][pl.ds(g * 16, 16)])
                    plsc.store_scatter(
                        rows_v,
                        [rows, jnp.full((16,), D, jnp.int32)],
                        b_v[pl.ds(g * 16, 16)])
                    for z in range(D + 1, RW):
                        plsc.store_scatter(
                            rows_v,
                            [rows, jnp.full((16,), z, jnp.int32)],
                            zeros)
                    return carry

                lax.fori_loop(0, TB_CH // 16, body, 0)
                pltpu.sync_copy(rows_v, tab_hbm.at[pl.ds(r0, TB_CH)])

    return k(emb_wT, emb_b)


NSPLIT = 4
NI_S = NI // NSPLIT
PER_W_S = NI_S // NW
NSTEP_S = PER_W_S // STEP
GT = (PER_W_S - NSTEP_S * STEP) // IPG


def _sc_gather(tab, idx_flat, base):
    @functools.partial(
        pl.kernel, mesh=plsc.VectorSubcoreMesh(**_MESH),
        compiler_params=_CP,
        out_type=jax.ShapeDtypeStruct((NI_S, RW), jnp.float32),
        scratch_types=[
            pltpu.VMEM((STEP,), jnp.int32),
            pltpu.VMEM((STEP, RW), jnp.float32),
            pltpu.SemaphoreType.DMA,
        ],
    )
    def k(tab_hbm, idx_hbm, out_hbm, idx_v, rows_v, sem):
        wid = lax.axis_index("s") * NC + lax.axis_index("c")

        def batch(o0, n):
            pltpu.sync_copy(idx_hbm.at[pl.ds(base + wid * PER_W_S + o0,
                                             n * IPG)],
                            idx_v.at[pl.ds(0, n * IPG)])
            copies = [
                pltpu.async_copy(tab_hbm.at[idx_v.at[pl.ds(j * IPG, IPG)]],
                                 rows_v.at[pl.ds(j * IPG, IPG)], sem)
                for j in range(n)
            ]
            for cp in copies:
                cp.wait()
            pltpu.sync_copy(rows_v.at[pl.ds(0, n * IPG)],
                            out_hbm.at[pl.ds(wid * PER_W_S + o0, n * IPG)])

        def body(c, carry):
            batch(c * STEP, G)
            return carry

        lax.fori_loop(0, NSTEP_S, body, 0)
        batch(NSTEP_S * STEP, GT)

    return k(tab, idx_flat)


BLK = 2048
NB = B // BLK


def _mlp_body(x_ref, w1_ref, b1_ref, w2_ref, b2_ref, w3_ref, out_ref):
    y = jnp.dot(x_ref[...], w1_ref[...], preferred_element_type=jnp.float32)
    col = lax.broadcasted_iota(jnp.int32, (BLK, 128), 1)
    h1 = jnp.where(col < H1, jnp.maximum(y + b1_ref[...], 0.0), 0.0)
    y2 = jnp.dot(h1, w2_ref[...], preferred_element_type=jnp.float32)
    h2 = jnp.where(col < H2, jnp.maximum(y2 + b2_ref[...], 0.0),
                   jnp.where(col == H2, 1.0, 0.0))
    y3 = jnp.dot(h2, w3_ref[...], preferred_element_type=jnp.float32)
    out_ref[...] = y3[:, 0] + y[:, H1]


def _tc_mlp(x, w1p, b1p, w2p, b2p, w3p):
    rows = x.shape[0]
    return pl.pallas_call(
        _mlp_body,
        grid=(rows // BLK,),
        in_specs=[
            pl.BlockSpec((BLK, XW), lambda i: (i, 0)),
            pl.BlockSpec((XW, 128), lambda i: (0, 0)),
            pl.BlockSpec((1, 128), lambda i: (0, 0)),
            pl.BlockSpec((128, 128), lambda i: (0, 0)),
            pl.BlockSpec((1, 128), lambda i: (0, 0)),
            pl.BlockSpec((128, 128), lambda i: (0, 0)),
        ],
        out_specs=pl.BlockSpec((BLK,), lambda i: (i,)),
        out_shape=jax.ShapeDtypeStruct((rows,), jnp.float32),
    )(x, w1p, b1p, w2p, b2p, w3p)


def _pack_weights(W1, b1, W2, b2, W3, b3):
    w1r = jnp.transpose(W1.reshape(H1, F, D), (1, 2, 0))
    w1p = (jnp.zeros((F, RW, 128), jnp.float32)
           .at[:, :D, :H1].set(w1r)
           .at[:, D, H1].set(1.0)
           .reshape(XW, 128))
    b1p = jnp.zeros((1, 128), jnp.float32).at[0, :H1].set(b1)
    w2p = jnp.zeros((128, 128), jnp.float32).at[:H1, :H2].set(W2.T)
    b2p = jnp.zeros((1, 128), jnp.float32).at[0, :H2].set(b2)
    w3p = (jnp.zeros((128, 128), jnp.float32)
           .at[:H2, 0].set(W3[0]).at[H2, 0].set(b3[0]))
    return w1p, b1p, w2p, b2p, w3p


def kernel(fids_batch, emb_w, emb_b, W1, b1, W2, b2, W3, b3):
    tab = _sc_build_tab(emb_w.T, emb_b)
    idx_flat = fids_batch.reshape(NI)
    w1p, b1p, w2p, b2p, w3p = _pack_weights(W1, b1, W2, b2, W3, b3)
    preds = []
    for h in range(NSPLIT):
        g = _sc_gather(tab, idx_flat, h * NI_S)
        x = g.reshape(B // NSPLIT, XW)
        preds.append(_tc_mlp(x, w1p, b1p, w2p, b2p, w3p))
    return jnp.concatenate(preds)
